# baseline (speedup 1.0000x reference)
import functools

import jax
import jax.numpy as jnp
from jax import lax
from jax.experimental import pallas as pl
from jax.experimental.pallas import tpu as pltpu
from jax.experimental.pallas import tpu_sc as plsc

N_TOK = 2048
N_E = 1024
E_DIM = 256
ALPHA = 0.9

K_CAND = 8
BI = 256
NBI = N_TOK // BI
NC = E_DIM // 8


def _score_body(x_ref, wt_ref, cand_ref, wn_ref):
    b = pl.program_id(0)
    xb = x_ref[...]
    wtb = wt_ref[...]

    @pl.when(b == 0)
    def _():
        wn_ref[...] = jnp.sum(wtb * wtb, axis=0, keepdims=True)

    mm = lax.dot_general(xb, wtb, (((1,), (0,)), ((), ())),
                         preferred_element_type=jnp.float32,
                         precision=lax.Precision.HIGHEST)
    score = wn_ref[...] - (mm + mm)
    iota = lax.broadcasted_iota(jnp.int32, (BI, N_E), 1)
    keys = (lax.bitcast_convert_type(score + 1.0, jnp.int32) &
            jnp.int32(~1023)) | iota
    cols = []
    for _ in range(K_CAND):
        m = jnp.min(keys, axis=1)
        cols.append((m & jnp.int32(1023))[:, None])
        keys = jnp.where(keys == m[:, None], jnp.int32(2**31 - 1), keys)
    cand_ref[...] = jnp.concatenate(cols, axis=1)


def _topk_scores(x, wt):
    return pl.pallas_call(
        _score_body,
        grid=(NBI,),
        in_specs=[
            pl.BlockSpec((BI, E_DIM), lambda b: (b, 0)),
            pl.BlockSpec((E_DIM, N_E), lambda b: (0, 0)),
        ],
        out_specs=pl.BlockSpec((BI, K_CAND), lambda b: (b, 0)),
        out_shape=jax.ShapeDtypeStruct((N_TOK, K_CAND), jnp.int32),
        scratch_shapes=[pltpu.VMEM((1, N_E), jnp.float32)],
    )(x, wt)


def _tree_body(xt_ref, wgt_ref, candt_ref, j_ref, loss_ref, dis_ref):
    k = pl.program_id(0)
    xb = xt_ref[...]
    wb = wgt_ref[...]

    acc = jnp.zeros((8, N_TOK), jnp.float32)
    for c in range(NC):
        row = slice(8 * c, 8 * c + 8)
        d = wb[row, :] - xb[row, :]
        t = d * d
        u = t + jnp.roll(t, 4, axis=0)
        v = u + jnp.roll(u, 2, axis=0)
        w = v + jnp.roll(v, 1, axis=0)
        acc = acc + w
    dis_ref[pl.ds(k, 1), :] = acc[0:1, :]

    @pl.when(k == K_CAND - 1)
    def _():
        dis = dis_ref[...]
        mv = jnp.min(dis, axis=0)
        hit = jnp.where(dis == mv[None, :], candt_ref[...], jnp.int32(N_E))
        j_ref[...] = jnp.min(hit, axis=0)
        loss_ref[0, 0] = jnp.sum(mv) * ((1.0 + ALPHA) / N_TOK)


def _tree_argmin(xt, wgt, candt):
    return pl.pallas_call(
        _tree_body,
        grid=(K_CAND,),
        in_specs=[
            pl.BlockSpec((E_DIM, N_TOK), lambda k: (0, 0)),
            pl.BlockSpec((E_DIM, N_TOK), lambda k: (0, k)),
            pl.BlockSpec((K_CAND, N_TOK), lambda k: (0, 0)),
        ],
        out_specs=[
            pl.BlockSpec((N_TOK,), lambda k: (0,)),
            pl.BlockSpec(memory_space=pltpu.SMEM, block_shape=(1, 1),
                         index_map=lambda k: (0, 0)),
        ],
        out_shape=[
            jax.ShapeDtypeStruct((N_TOK,), jnp.int32),
            jax.ShapeDtypeStruct((1, 1), jnp.float32),
        ],
        scratch_shapes=[pltpu.VMEM((K_CAND, N_TOK), jnp.float32)],
    )(xt, wgt, candt)


def _sc_gather(W, j, rows_per_chunk=64):
    B = j.shape[0]
    info = plsc.get_sparse_core_info()
    ncores, nsub = info.num_cores, info.num_subcores
    nw = ncores * nsub
    bpw = B // nw
    nch = bpw // rows_per_chunk
    mesh = plsc.VectorSubcoreMesh(core_axis_name="c", subcore_axis_name="s")

    @functools.partial(
        pl.kernel,
        mesh=mesh,
        out_type=jax.ShapeDtypeStruct((B, E_DIM), jnp.float32),
        scratch_types=[
            pltpu.VMEM((bpw,), jnp.int32),
            pltpu.VMEM((rows_per_chunk, E_DIM), jnp.float32),
            pltpu.VMEM((rows_per_chunk, E_DIM), jnp.float32),
            pltpu.SemaphoreType.DMA,
            pltpu.SemaphoreType.DMA,
            pltpu.SemaphoreType.DMA,
            pltpu.SemaphoreType.DMA,
        ],
    )
    def gather_k(w_hbm, idx_hbm, out_hbm, idx_v, rv0, rv1, g0, g1, w0, w1):
        wid = lax.axis_index("s") * ncores + lax.axis_index("c")
        base = wid * bpw
        pltpu.sync_copy(idx_hbm.at[pl.ds(base, bpw)], idx_v)
        bufs = (rv0, rv1)
        gsems = (g0, g1)
        wsems = (w0, w1)
        gops = [None] * nch
        wops = [None] * nch
        for h in range(nch):
            c = h % 2
            if h >= 2:
                wops[h - 2].wait()
            gops[h] = pltpu.async_copy(
                w_hbm.at[idx_v.at[pl.ds(h * rows_per_chunk,
                                        rows_per_chunk)]],
                bufs[c], gsems[c])
            gops[h].wait()
            wops[h] = pltpu.async_copy(
                bufs[c],
                out_hbm.at[pl.ds(base + h * rows_per_chunk,
                                 rows_per_chunk)],
                wsems[c])
        for h in range(max(0, nch - 2), nch):
            wops[h].wait()

    return gather_k(W, j)


def kernel(x, W):
    xt = x.T
    wt = W.T
    cand = _topk_scores(x, wt)
    candt = cand.T
    wg = _sc_gather(W, candt.reshape(-1), rows_per_chunk=128)
    wgt = wg.T
    j, loss = _tree_argmin(xt, wgt, candt)
    W_j = _sc_gather(W, j, rows_per_chunk=64)
    return (W_j, loss.reshape(()))
